# baseline (speedup 1.0000x reference)
import jax
import jax.numpy as jnp
from jax import lax
from jax.experimental import pallas as pl
from jax.experimental.pallas import tpu as pltpu

N = 10000
M = 5000
D = 128
BR = 200
GRID = N // BR
MP = M + 8
SEARCH_ITERS = 13


def _searchsorted(idx_smem, target):
    def step(_, lohi):
        lo, hi = lohi
        mid = (lo + hi) // 2
        below = idx_smem[mid] < target
        return (jnp.where(below, mid + 1, lo), jnp.where(below, hi, mid))

    lo, _ = lax.fori_loop(0, SEARCH_ITERS, step, (0, M))
    return lo


def _body(idx_smem, a_ref, x_ref, ao_ref, nx_ref):
    i = pl.program_id(0)
    ao_ref[...] = a_ref[...]
    nx_ref[...] = jnp.zeros((BR, D), jnp.float32)

    row0 = i * BR
    p_lo = _searchsorted(idx_smem, row0)
    p_hi = _searchsorted(idx_smem, row0 + BR)

    def place(p, _):
        local = idx_smem[p] - row0
        nx_ref[pl.ds(local, 1), :] = x_ref[pl.ds(p, 1), :]
        return 0

    lax.fori_loop(p_lo, p_hi, place, 0)


@jax.jit
def _fused(A, X, idx):
    return pl.pallas_call(
        _body,
        grid=(GRID,),
        in_specs=[
            pl.BlockSpec(memory_space=pltpu.SMEM),
            pl.BlockSpec((BR, N), lambda i: (i, 0)),
            pl.BlockSpec((M, D), lambda i: (0, 0)),
        ],
        out_specs=[
            pl.BlockSpec((BR, N), lambda i: (i, 0)),
            pl.BlockSpec((BR, D), lambda i: (i, 0)),
        ],
        out_shape=[
            jax.ShapeDtypeStruct((N, N), jnp.float32),
            jax.ShapeDtypeStruct((N, D), jnp.float32),
        ],
    )(idx, A, X)


def kernel(A, X, idx):
    idx_pad = jnp.concatenate(
        [idx.astype(jnp.int32),
         jnp.full((MP - M,), jnp.iinfo(jnp.int32).max, jnp.int32)])
    a_out, new_x = _fused(A, X, idx_pad)
    return (a_out, new_x)


# speedup vs baseline: 1.0061x; 1.0061x over previous
import jax
import jax.numpy as jnp
from jax import lax
from jax.experimental import pallas as pl
from jax.experimental.pallas import tpu as pltpu

N = 10000
M = 5000
D = 128
BR = 200
GRID = N // BR
PPS = M // GRID


def _body(idx_smem, a_ref, x_ref, ao_ref, nx_ref):
    i = pl.program_id(0)
    ao_ref[...] = a_ref[...]

    @pl.when(i == 0)
    def _():
        nx_ref[...] = jnp.zeros((N, D), jnp.float32)

    def place(k, _):
        p = i * PPS + k
        row = idx_smem[p]
        nx_ref[pl.ds(row, 1), :] = x_ref[pl.ds(p, 1), :]
        return 0

    lax.fori_loop(0, PPS, place, 0)


@jax.jit
def _fused(A, X, idx):
    return pl.pallas_call(
        _body,
        grid=(GRID,),
        in_specs=[
            pl.BlockSpec(memory_space=pltpu.SMEM),
            pl.BlockSpec((BR, N), lambda i: (i, 0)),
            pl.BlockSpec((M, D), lambda i: (0, 0)),
        ],
        out_specs=[
            pl.BlockSpec((BR, N), lambda i: (i, 0)),
            pl.BlockSpec((N, D), lambda i: (0, 0)),
        ],
        out_shape=[
            jax.ShapeDtypeStruct((N, N), jnp.float32),
            jax.ShapeDtypeStruct((N, D), jnp.float32),
        ],
    )(idx, A, X)


def kernel(A, X, idx):
    a_out, new_x = _fused(A, X, idx.astype(jnp.int32))
    return (a_out, new_x)
